# Initial kernel scaffold; baseline (speedup 1.0000x reference)
import functools

import jax
import jax.numpy as jnp
from jax import lax
from jax.experimental import pallas as pl
from jax.experimental.pallas import tpu as pltpu
from jax.experimental.pallas import tpu_sc as plsc

N_NODES = 10000
N_EDGES = 320000
DIM = 128
OUT_DIM = 10

NC = 2
NS = 16
NW = NC * NS
K = 128
NCHUNK = -(-N_EDGES // (NW * K))
E_PAD = NW * K * NCHUNK
N_PAD = 10240
RPT = N_PAD // NS


def _agg_body(y_hbm, se_hbm, de_hbm, zero_hbm, out_hbm, acc, idx_v, rows_v,
              sem):
    c = lax.axis_index("c")
    s = lax.axis_index("s")
    wid = c * NS + s
    pltpu.sync_copy(zero_hbm.at[pl.ds(s * RPT, RPT)],
                    acc.at[pl.ds(s * RPT, RPT)])
    plsc.subcore_barrier()

    def body(i, carry):
        pltpu.sync_copy(se_hbm.at[wid, i], idx_v)
        pltpu.async_copy(y_hbm.at[idx_v], rows_v, sem).wait()
        pltpu.sync_copy(de_hbm.at[wid, i], idx_v)
        pltpu.sync_copy(rows_v, acc.at[idx_v], add=True)
        return carry

    lax.fori_loop(0, NCHUNK, body, 0)
    plsc.subcore_barrier()
    pltpu.sync_copy(acc.at[pl.ds(s * RPT, RPT)],
                    out_hbm.at[c, pl.ds(s * RPT, RPT)])


_agg = pl.kernel(
    _agg_body,
    out_type=jax.ShapeDtypeStruct((NC, N_PAD, DIM), jnp.float32),
    mesh=plsc.VectorSubcoreMesh(core_axis_name="c", subcore_axis_name="s"),
    scratch_types=[
        pltpu.VMEM_SHARED((N_PAD, DIM), jnp.float32),
        pltpu.VMEM((K,), jnp.int32),
        pltpu.VMEM((K, DIM), jnp.float32),
        pltpu.SemaphoreType.DMA,
    ],
)


def _mm_body(x_ref, w_ref, o_ref):
    o_ref[...] = jnp.dot(x_ref[...], w_ref[...],
                         preferred_element_type=jnp.float32)


def _fuse_body(p_ref, b_ref, w_ref, o_ref):
    h = p_ref[0, :N_NODES, :] + p_ref[1, :N_NODES, :] + b_ref[...]
    h = jnp.where(h > 0.0, h, jnp.exp(h) - 1.0)
    o_ref[...] = jnp.dot(h, w_ref[...], preferred_element_type=jnp.float32)


def _head_body(p_ref, b_ref, wc_ref, bc_ref, o_ref):
    h = p_ref[0, :N_NODES, :] + p_ref[1, :N_NODES, :] + b_ref[...]
    h = jnp.where(h > 0.0, h, jnp.exp(h) - 1.0)
    hg = jnp.sum(h, axis=0, keepdims=True) * (1.0 / N_NODES)
    o_ref[...] = (jnp.dot(hg, wc_ref[...], preferred_element_type=jnp.float32)
                  + bc_ref[...])


def _mm(x, w):
    return pl.pallas_call(
        _mm_body,
        out_shape=jax.ShapeDtypeStruct((x.shape[0], DIM), jnp.float32),
    )(x, w)


def _fuse(parts, b, w):
    return pl.pallas_call(
        _fuse_body,
        out_shape=jax.ShapeDtypeStruct((N_NODES, DIM), jnp.float32),
    )(parts, b.reshape(1, DIM), w)


def _head(parts, b, wc_pad, bc_pad):
    return pl.pallas_call(
        _head_body,
        out_shape=jax.ShapeDtypeStruct((1, DIM), jnp.float32),
    )(parts, b.reshape(1, DIM), wc_pad, bc_pad)


def kernel(x, edge_index, W1, b1, W2, b2, W3, b3, Wc, bc):
    src = edge_index[0].astype(jnp.int32)
    dst = edge_index[1].astype(jnp.int32)
    pad = E_PAD - N_EDGES
    se = jnp.concatenate([src, jnp.zeros((pad,), jnp.int32)])
    de = jnp.concatenate([dst, jnp.full((pad,), N_NODES, jnp.int32)])
    se = se.reshape(NW, NCHUNK, K)
    de = de.reshape(NW, NCHUNK, K)
    zero = jnp.zeros((N_PAD, DIM), jnp.float32)
    wc_pad = jnp.pad(Wc, ((0, 0), (0, DIM - OUT_DIM)))
    bc_pad = jnp.pad(bc, (0, DIM - OUT_DIM)).reshape(1, DIM)

    y = _mm(x, W1)
    p = _agg(y, se, de, zero)
    y = _fuse(p, b1, W2)
    p = _agg(y, se, de, zero)
    y = _fuse(p, b2, W3)
    p = _agg(y, se, de, zero)
    out = _head(p, b3, wc_pad, bc_pad)
    return out[:, :OUT_DIM]


# speedup vs baseline: 2.9365x; 2.9365x over previous
import functools

import jax
import jax.numpy as jnp
from jax import lax
from jax.experimental import pallas as pl
from jax.experimental.pallas import tpu as pltpu
from jax.experimental.pallas import tpu_sc as plsc

N_NODES = 10000
N_EDGES = 320000
DIM = 128
OUT_DIM = 10

NC = 2
NS = 16
NW = NC * NS
K = 128
NCHUNK = -(-N_EDGES // (NW * K))
NCHUNK += NCHUNK % 2
E_PAD = NW * K * NCHUNK
N_PAD = 10240
RPT = N_PAD // NS


def _agg_body(y_hbm, ei_hbm, zero_hbm, out_hbm, acc,
              idx0, idx1, rows0, rows1, gsem0, gsem1, ssem0, ssem1):
    c = lax.axis_index("c")
    s = lax.axis_index("s")
    wid = c * NS + s
    idx = (idx0, idx1)
    rows = (rows0, rows1)
    gsem = (gsem0, gsem1)
    ssem = (ssem0, ssem1)

    pltpu.sync_copy(zero_hbm.at[pl.ds(s * RPT, RPT)],
                    acc.at[pl.ds(s * RPT, RPT)])
    plsc.subcore_barrier()

    def fetch(i, b):
        pltpu.sync_copy(ei_hbm.at[wid, i], idx[b])
        pltpu.async_copy(y_hbm.at[idx[b].at[0]], rows[b], gsem[b])

    def scat(i, b):
        pltpu.make_async_copy(y_hbm.at[idx[b].at[0]], rows[b], gsem[b]).wait()
        pltpu.async_copy(rows[b], acc.at[idx[b].at[1]], ssem[b], add=True)

    def wait_scat(b):
        pltpu.make_async_copy(rows[b], acc.at[idx[b].at[1]], ssem[b]).wait()

    fetch(0, 0)

    def body(j, carry):
        for b in range(2):
            i = 2 * j + b
            if b == 0:
                @pl.when(j > 0)
                def _():
                    wait_scat(1)

                fetch(i + 1, 1)
            else:
                wait_scat(0)

                @pl.when(j < NCHUNK // 2 - 1)
                def _():
                    fetch(i + 1, 0)
            scat(i, b)
        return carry

    lax.fori_loop(0, NCHUNK // 2, body, 0)
    wait_scat(1)

    plsc.subcore_barrier()
    pltpu.sync_copy(acc.at[pl.ds(s * RPT, RPT)],
                    out_hbm.at[c, pl.ds(s * RPT, RPT)])


_agg = pl.kernel(
    _agg_body,
    out_type=jax.ShapeDtypeStruct((NC, N_PAD, DIM), jnp.float32),
    mesh=plsc.VectorSubcoreMesh(core_axis_name="c", subcore_axis_name="s",
                                num_cores=NC, num_subcores=NS),
    scratch_types=[
        pltpu.VMEM_SHARED((N_PAD, DIM), jnp.float32),
        pltpu.VMEM((2, K), jnp.int32),
        pltpu.VMEM((2, K), jnp.int32),
        pltpu.VMEM((K, DIM), jnp.float32),
        pltpu.VMEM((K, DIM), jnp.float32),
        pltpu.SemaphoreType.DMA,
        pltpu.SemaphoreType.DMA,
        pltpu.SemaphoreType.DMA,
        pltpu.SemaphoreType.DMA,
    ],
)


def _mm_body(x_ref, w_ref, o_ref):
    o_ref[...] = jnp.dot(x_ref[...], w_ref[...],
                         preferred_element_type=jnp.float32)


def _fuse_body(p_ref, b_ref, w_ref, o_ref):
    h = p_ref[0, :N_NODES, :] + p_ref[1, :N_NODES, :] + b_ref[...]
    h = jnp.where(h > 0.0, h, jnp.exp(h) - 1.0)
    o_ref[...] = jnp.dot(h, w_ref[...], preferred_element_type=jnp.float32)


def _head_body(p_ref, b_ref, wc_ref, bc_ref, o_ref):
    h = p_ref[0, :N_NODES, :] + p_ref[1, :N_NODES, :] + b_ref[...]
    h = jnp.where(h > 0.0, h, jnp.exp(h) - 1.0)
    hg = jnp.sum(h, axis=0, keepdims=True) * (1.0 / N_NODES)
    o_ref[...] = (jnp.dot(hg, wc_ref[...], preferred_element_type=jnp.float32)
                  + bc_ref[...])


def _mm(x, w):
    return pl.pallas_call(
        _mm_body,
        out_shape=jax.ShapeDtypeStruct((x.shape[0], DIM), jnp.float32),
    )(x, w)


def _fuse(parts, b, w):
    return pl.pallas_call(
        _fuse_body,
        out_shape=jax.ShapeDtypeStruct((N_NODES, DIM), jnp.float32),
    )(parts, b.reshape(1, DIM), w)


def _head(parts, b, wc_pad, bc_pad):
    return pl.pallas_call(
        _head_body,
        out_shape=jax.ShapeDtypeStruct((1, DIM), jnp.float32),
    )(parts, b.reshape(1, DIM), wc_pad, bc_pad)


def kernel(x, edge_index, W1, b1, W2, b2, W3, b3, Wc, bc):
    src = edge_index[0].astype(jnp.int32)
    dst = edge_index[1].astype(jnp.int32)
    pad = E_PAD - N_EDGES
    se = jnp.concatenate([src, jnp.zeros((pad,), jnp.int32)])
    de = jnp.concatenate([dst, jnp.full((pad,), N_NODES, jnp.int32)])
    ei = jnp.stack([se.reshape(NW, NCHUNK, K),
                    de.reshape(NW, NCHUNK, K)], axis=2)
    zero = jnp.zeros((N_PAD, DIM), jnp.float32)
    wc_pad = jnp.pad(Wc, ((0, 0), (0, DIM - OUT_DIM)))
    bc_pad = jnp.pad(bc, (0, DIM - OUT_DIM)).reshape(1, DIM)

    y = _mm(x, W1)
    p = _agg(y, ei, zero)
    y = _fuse(p, b1, W2)
    p = _agg(y, ei, zero)
    y = _fuse(p, b2, W3)
    p = _agg(y, ei, zero)
    out = _head(p, b3, wc_pad, bc_pad)
    return out[:, :OUT_DIM]
